# baseline (speedup 1.0000x reference)
import jax
import jax.numpy as jnp
from jax.experimental import pallas as pl
from jax.experimental.pallas import tpu as pltpu

D_MODEL = 1024
D_FF = 4096
E = 8
R = 2
ER = E * R
SCALE = 4.0 / R
SEQ = 2048
FT = 512
NF = D_FF // FT
NEG = -1e30


def _moe_kernel(x_ref, Wr_ref, br_ref, W1_ref, b1_ref, W2_ref,
                A1f_ref, B1f_ref, A2f_ref, B2f_ref, b2_ref, Wh_ref, bh_ref,
                out_ref,
                C0, C1, sel0, sel1, g0s, g1s, t2a0, t2a1, accd,
                xbf, u0b, u1b):
    fi = pl.program_id(0)

    @pl.when(fi == 0)
    def _():
        xv = x_ref[...].astype(jnp.bfloat16)
        xbf[...] = xv
        logits = jnp.dot(xv, Wr_ref[...].astype(jnp.bfloat16),
                         preferred_element_type=jnp.float32) + br_ref[...]
        ecol = jax.lax.broadcasted_iota(jnp.int32, (SEQ, E), 1)
        m0 = jnp.max(logits, axis=1, keepdims=True)
        i0 = jnp.min(jnp.where(logits == m0, ecol, E), axis=1, keepdims=True)
        l2 = jnp.where(ecol == i0, NEG, logits)
        m1 = jnp.max(l2, axis=1, keepdims=True)
        i1 = jnp.min(jnp.where(l2 == m1, ecol, E), axis=1, keepdims=True)
        e1 = jnp.exp(m1 - m0)
        g0 = 1.0 / (1.0 + e1)
        g0s[...] = g0
        g1s[...] = e1 * g0
        t1all = jnp.dot(xv, A1f_ref[...].astype(jnp.bfloat16),
                        preferred_element_type=jnp.float32)
        kcol = jax.lax.broadcasted_iota(jnp.int32, (SEQ, ER), 1) // R
        s0 = (kcol == i0).astype(jnp.float32)
        s1 = (kcol == i1).astype(jnp.float32)
        sel0[...] = s0
        sel1[...] = s1
        C0[...] = t1all * s0
        C1[...] = t1all * s1
        t2a0[...] = jnp.zeros_like(t2a0)
        t2a1[...] = jnp.zeros_like(t2a1)
        accd[...] = jnp.zeros_like(accd)
        u0b[1] = jnp.zeros_like(u0b[1])
        u1b[1] = jnp.zeros_like(u1b[1])

    slot_p = jax.lax.rem(fi, 2)
    base = (jnp.dot(xbf[...], W1_ref[...].astype(jnp.bfloat16),
                    preferred_element_type=jnp.float32) + b1_ref[...])
    b1f = B1f_ref[...].astype(jnp.bfloat16)
    l1_0 = jnp.dot(C0[...].astype(jnp.bfloat16), b1f,
                   preferred_element_type=jnp.float32) * SCALE
    l1_1 = jnp.dot(C1[...].astype(jnp.bfloat16), b1f,
                   preferred_element_type=jnp.float32) * SCALE
    u0b[slot_p] = (base + l1_0).astype(jnp.bfloat16)
    u1b[slot_p] = (base + l1_1).astype(jnp.bfloat16)

    slot_c = jax.lax.rem(fi + 1, 2)
    h0 = jax.nn.gelu(u0b[slot_c])
    h1 = jax.nn.gelu(u1b[slot_c])
    comb = (g0s[...].astype(jnp.bfloat16) * h0
            + g1s[...].astype(jnp.bfloat16) * h1)
    ones_row = jnp.full((1, SEQ), 1.0, dtype=jnp.bfloat16)
    chs = jnp.dot(ones_row, comb, preferred_element_type=jnp.float32)
    w2r = W2_ref[...].astype(jnp.bfloat16).astype(jnp.float32)
    accd[...] += jnp.dot(chs, w2r, precision=jax.lax.Precision.HIGHEST,
                         preferred_element_type=jnp.float32)
    a2f = A2f_ref[...].astype(jnp.bfloat16)
    t2a0[...] += jnp.dot(h0, a2f, preferred_element_type=jnp.float32)
    t2a1[...] += jnp.dot(h1, a2f, preferred_element_type=jnp.float32)

    @pl.when(fi == NF)
    def _():
        m0 = t2a0[...] * sel0[...] * (g0s[...] * SCALE)
        m1 = t2a1[...] * sel1[...] * (g1s[...] * SCALE)
        dsum = jnp.sum(m0 + m1, axis=0, keepdims=True)
        b2r = B2f_ref[...].astype(jnp.bfloat16).astype(jnp.float32)
        tot = accd[...] + jnp.dot(dsum, b2r,
                                  precision=jax.lax.Precision.HIGHEST,
                                  preferred_element_type=jnp.float32)
        pooled = tot * (1.0 / SEQ) + b2_ref[...]
        out_ref[...] = jnp.dot(pooled.astype(jnp.bfloat16),
                               Wh_ref[...].astype(jnp.bfloat16),
                               preferred_element_type=jnp.float32) + bh_ref[...]


def _clamp_hi(i):
    return jnp.minimum(i, NF - 1)


def _clamp_lo(i):
    return jnp.maximum(i - 1, 0)


def kernel(x, Wr, br, W1, b1, W2, b2, A1, B1, A2, B2, Wh, bh):
    B, S, D = x.shape
    xf = x.reshape(S, D)
    A1f = A1.transpose(1, 0, 2).reshape(D_MODEL, ER)
    B1f = B1.reshape(ER, D_FF)
    A2f = A2.transpose(1, 0, 2).reshape(D_FF, ER)
    B2f = B2.reshape(ER, D_MODEL)

    out = pl.pallas_call(
        _moe_kernel,
        grid=(NF + 1,),
        in_specs=[
            pl.BlockSpec((SEQ, D_MODEL), lambda fi: (0, 0)),
            pl.BlockSpec((D_MODEL, E), lambda fi: (0, 0)),
            pl.BlockSpec((1, E), lambda fi: (0, 0)),
            pl.BlockSpec((D_MODEL, FT), lambda fi: (0, _clamp_hi(fi))),
            pl.BlockSpec((1, FT), lambda fi: (0, _clamp_hi(fi))),
            pl.BlockSpec((FT, D_MODEL), lambda fi: (_clamp_lo(fi), 0)),
            pl.BlockSpec((D_MODEL, ER), lambda fi: (0, 0)),
            pl.BlockSpec((ER, FT), lambda fi: (0, _clamp_hi(fi))),
            pl.BlockSpec((FT, ER), lambda fi: (_clamp_lo(fi), 0)),
            pl.BlockSpec((ER, D_MODEL), lambda fi: (0, 0)),
            pl.BlockSpec((1, D_MODEL), lambda fi: (0, 0)),
            pl.BlockSpec((D_MODEL, 2), lambda fi: (0, 0)),
            pl.BlockSpec((1, 2), lambda fi: (0, 0)),
        ],
        out_specs=pl.BlockSpec((1, 2), lambda fi: (0, 0)),
        out_shape=jax.ShapeDtypeStruct((1, 2), jnp.float32),
        scratch_shapes=[
            pltpu.VMEM((SEQ, ER), jnp.float32),
            pltpu.VMEM((SEQ, ER), jnp.float32),
            pltpu.VMEM((SEQ, ER), jnp.float32),
            pltpu.VMEM((SEQ, ER), jnp.float32),
            pltpu.VMEM((SEQ, 1), jnp.float32),
            pltpu.VMEM((SEQ, 1), jnp.float32),
            pltpu.VMEM((SEQ, ER), jnp.float32),
            pltpu.VMEM((SEQ, ER), jnp.float32),
            pltpu.VMEM((1, D_MODEL), jnp.float32),
            pltpu.VMEM((SEQ, D_MODEL), jnp.bfloat16),
            pltpu.VMEM((2, SEQ, FT), jnp.bfloat16),
            pltpu.VMEM((2, SEQ, FT), jnp.bfloat16),
        ],
        compiler_params=pltpu.CompilerParams(
            dimension_semantics=("arbitrary",),
        ),
    )(xf, Wr, br.reshape(1, E), W1, b1.reshape(1, D_FF), W2,
      A1f, B1f, A2f, B2f, b2.reshape(1, D_MODEL), Wh, bh.reshape(1, 2))

    return out.reshape(B, 2)


# speedup vs baseline: 1.2080x; 1.2080x over previous
import jax
import jax.numpy as jnp
from jax.experimental import pallas as pl
from jax.experimental.pallas import tpu as pltpu

D_MODEL = 1024
D_FF = 4096
E = 8
R = 2
ER = E * R
SCALE = 4.0 / R
SEQ = 2048
FT = 512
NF = D_FF // FT
NEG = -1e30


def _moe_kernel(x_ref, Wr_ref, br_ref, W1_ref, b1_ref, W2_ref,
                A1f_ref, B1f_ref, A2f_ref, B2f_ref, b2_ref, Wh_ref, bh_ref,
                out_ref,
                C0, C1, sel0, sel1, g0s, g1s, t2a0, t2a1, accd, xbf):
    fi = pl.program_id(0)

    @pl.when(fi == 0)
    def _():
        xv = x_ref[...].astype(jnp.bfloat16)
        xbf[...] = xv
        logits = jnp.dot(xv, Wr_ref[...].astype(jnp.bfloat16),
                         preferred_element_type=jnp.float32) + br_ref[...]
        ecol = jax.lax.broadcasted_iota(jnp.int32, (SEQ, E), 1)
        m0 = jnp.max(logits, axis=1, keepdims=True)
        i0 = jnp.min(jnp.where(logits == m0, ecol, E), axis=1, keepdims=True)
        l2 = jnp.where(ecol == i0, NEG, logits)
        m1 = jnp.max(l2, axis=1, keepdims=True)
        i1 = jnp.min(jnp.where(l2 == m1, ecol, E), axis=1, keepdims=True)
        e1 = jnp.exp(m1 - m0)
        g0 = 1.0 / (1.0 + e1)
        g0s[...] = g0
        g1s[...] = e1 * g0
        t1all = jnp.dot(xv, A1f_ref[...].astype(jnp.bfloat16),
                        preferred_element_type=jnp.float32)
        kcol = jax.lax.broadcasted_iota(jnp.int32, (SEQ, ER), 1) // R
        s0 = (kcol == i0).astype(jnp.float32)
        s1 = (kcol == i1).astype(jnp.float32)
        sel0[...] = s0
        sel1[...] = s1
        C0[...] = t1all * s0
        C1[...] = t1all * s1
        t2a0[...] = jnp.zeros_like(t2a0)
        t2a1[...] = jnp.zeros_like(t2a1)
        accd[...] = jnp.zeros_like(accd)

    base = (jnp.dot(xbf[...], W1_ref[...].astype(jnp.bfloat16),
                    preferred_element_type=jnp.float32) + b1_ref[...])
    b1f = B1f_ref[...].astype(jnp.bfloat16)
    l1_0 = jnp.dot(C0[...].astype(jnp.bfloat16), b1f,
                   preferred_element_type=jnp.float32) * SCALE
    l1_1 = jnp.dot(C1[...].astype(jnp.bfloat16), b1f,
                   preferred_element_type=jnp.float32) * SCALE
    h0 = jax.nn.gelu((base + l1_0).astype(jnp.bfloat16))
    h1 = jax.nn.gelu((base + l1_1).astype(jnp.bfloat16))
    comb = (g0s[...].astype(jnp.bfloat16) * h0
            + g1s[...].astype(jnp.bfloat16) * h1)
    ones_row = jnp.full((1, SEQ), 1.0, dtype=jnp.bfloat16)
    chs = jnp.dot(ones_row, comb, preferred_element_type=jnp.float32)
    w2b = W2_ref[...].astype(jnp.bfloat16)
    chs_hi = chs.astype(jnp.bfloat16)
    chs_lo = (chs - chs_hi.astype(jnp.float32)).astype(jnp.bfloat16)
    accd[...] += (jnp.dot(chs_hi, w2b, preferred_element_type=jnp.float32)
                  + jnp.dot(chs_lo, w2b, preferred_element_type=jnp.float32))
    a2f = A2f_ref[...].astype(jnp.bfloat16)
    t2a0[...] += jnp.dot(h0, a2f, preferred_element_type=jnp.float32)
    t2a1[...] += jnp.dot(h1, a2f, preferred_element_type=jnp.float32)

    @pl.when(fi == NF - 1)
    def _():
        m0 = t2a0[...] * sel0[...] * (g0s[...] * SCALE)
        m1 = t2a1[...] * sel1[...] * (g1s[...] * SCALE)
        dsum = jnp.sum(m0 + m1, axis=0, keepdims=True)
        b2b = B2f_ref[...].astype(jnp.bfloat16)
        ds_hi = dsum.astype(jnp.bfloat16)
        ds_lo = (dsum - ds_hi.astype(jnp.float32)).astype(jnp.bfloat16)
        tot = (accd[...]
               + jnp.dot(ds_hi, b2b, preferred_element_type=jnp.float32)
               + jnp.dot(ds_lo, b2b, preferred_element_type=jnp.float32))
        pooled = tot * (1.0 / SEQ) + b2_ref[...]
        out_ref[...] = jnp.dot(pooled.astype(jnp.bfloat16),
                               Wh_ref[...].astype(jnp.bfloat16),
                               preferred_element_type=jnp.float32) + bh_ref[...]


def kernel(x, Wr, br, W1, b1, W2, b2, A1, B1, A2, B2, Wh, bh):
    B, S, D = x.shape
    xf = x.reshape(S, D)
    A1f = A1.transpose(1, 0, 2).reshape(D_MODEL, ER)
    B1f = B1.reshape(ER, D_FF)
    A2f = A2.transpose(1, 0, 2).reshape(D_FF, ER)
    B2f = B2.reshape(ER, D_MODEL)

    out = pl.pallas_call(
        _moe_kernel,
        grid=(NF,),
        in_specs=[
            pl.BlockSpec((SEQ, D_MODEL), lambda fi: (0, 0)),
            pl.BlockSpec((D_MODEL, E), lambda fi: (0, 0)),
            pl.BlockSpec((1, E), lambda fi: (0, 0)),
            pl.BlockSpec((D_MODEL, FT), lambda fi: (0, fi)),
            pl.BlockSpec((1, FT), lambda fi: (0, fi)),
            pl.BlockSpec((FT, D_MODEL), lambda fi: (fi, 0)),
            pl.BlockSpec((D_MODEL, ER), lambda fi: (0, 0)),
            pl.BlockSpec((ER, FT), lambda fi: (0, fi)),
            pl.BlockSpec((FT, ER), lambda fi: (fi, 0)),
            pl.BlockSpec((ER, D_MODEL), lambda fi: (0, 0)),
            pl.BlockSpec((1, D_MODEL), lambda fi: (0, 0)),
            pl.BlockSpec((D_MODEL, 2), lambda fi: (0, 0)),
            pl.BlockSpec((1, 2), lambda fi: (0, 0)),
        ],
        out_specs=pl.BlockSpec((1, 2), lambda fi: (0, 0)),
        out_shape=jax.ShapeDtypeStruct((1, 2), jnp.float32),
        scratch_shapes=[
            pltpu.VMEM((SEQ, ER), jnp.float32),
            pltpu.VMEM((SEQ, ER), jnp.float32),
            pltpu.VMEM((SEQ, ER), jnp.float32),
            pltpu.VMEM((SEQ, ER), jnp.float32),
            pltpu.VMEM((SEQ, 1), jnp.float32),
            pltpu.VMEM((SEQ, 1), jnp.float32),
            pltpu.VMEM((SEQ, ER), jnp.float32),
            pltpu.VMEM((SEQ, ER), jnp.float32),
            pltpu.VMEM((1, D_MODEL), jnp.float32),
            pltpu.VMEM((SEQ, D_MODEL), jnp.bfloat16),
        ],
        compiler_params=pltpu.CompilerParams(
            dimension_semantics=("arbitrary",),
        ),
    )(xf, Wr, br.reshape(1, E), W1, b1.reshape(1, D_FF), W2,
      A1f, B1f, A2f, B2f, b2.reshape(1, D_MODEL), Wh, bh.reshape(1, 2))

    return out.reshape(B, 2)
